# Initial kernel scaffold; baseline (speedup 1.0000x reference)
import jax
import jax.numpy as jnp
from jax.experimental import pallas as pl


def kernel(x, edge_index, bn_gamma, bn_beta, W_enc, b_enc, W1, b1, ln1_g, ln1_b, W2, b2, ln2_g, ln2_b, Wr1, br1, Wr2, br2):
    raise NotImplementedError("write your pallas kernel here")


# speedup vs baseline: 23.7009x; 23.7009x over previous
import functools

import jax
import jax.numpy as jnp
from jax import lax
from jax.experimental import pallas as pl
from jax.experimental.pallas import tpu as pltpu
from jax.experimental.pallas import tpu_sc as plsc

NC = 2
NS = 16
NW = NC * NS
LANES = 16
CH = 128


def _count_body(np_, nch, dst_hbm, ones_hbm, zeros_hbm, cnt_out,
                idx_d, ones_v, cnt_sh):
    c = lax.axis_index("c")
    s = lax.axis_index("s")
    w = c * NS + s
    stripe = np_ // NS
    pltpu.sync_copy(dst_hbm.at[w], idx_d)
    pltpu.sync_copy(ones_hbm, ones_v)
    pltpu.sync_copy(zeros_hbm, cnt_sh.at[pl.ds(s * stripe, stripe)])
    plsc.subcore_barrier()

    def count_body(j, carry):
        pltpu.sync_copy(ones_v, cnt_sh.at[idx_d.at[j]], add=True)
        return carry

    lax.fori_loop(0, nch, count_body, 0)
    plsc.subcore_barrier()
    pltpu.sync_copy(cnt_sh.at[pl.ds(s * stripe, stripe)],
                    cnt_out.at[c, pl.ds(s * stripe, stripe)])


def _sc_count(dst3, ones8, zeros8, np_):
    nw, nch, ch = dst3.shape
    assert nw == NW and ch == CH
    mesh = plsc.VectorSubcoreMesh(core_axis_name="c", subcore_axis_name="s",
                                  num_cores=NC)
    return pl.kernel(
        functools.partial(_count_body, np_, nch),
        out_type=jax.ShapeDtypeStruct((NC, np_, 8), jnp.float32),
        mesh=mesh,
        scratch_types=[
            pltpu.VMEM((nch, CH), jnp.int32),
            pltpu.VMEM((CH, 8), jnp.float32),
            pltpu.VMEM_SHARED((np_, 8), jnp.float32),
        ],
        compiler_params=pltpu.CompilerParams(needs_layout_passes=False,
                                             use_tc_tiling_on_sc=False),
    )(dst3, ones8, zeros8)


def _scatter_body(np_, nch, h, g_hbm, src_hbm, dst_hbm, out_hbm,
                  idx_s, idx_d, rows, zbuf, g_sh, acc, gsem):
    c = lax.axis_index("c")
    s = lax.axis_index("s")
    w = c * NS + s
    stripe = np_ // NS

    pltpu.sync_copy(src_hbm.at[w], idx_s)
    pltpu.sync_copy(dst_hbm.at[w], idx_d)

    pltpu.sync_copy(g_hbm.at[pl.ds(s * stripe, stripe)],
                    g_sh.at[pl.ds(s * stripe, stripe)])
    zeros = jnp.zeros((LANES,), jnp.float32)

    def zero_body(i, carry):
        for k in range(h // LANES):
            zbuf[i, pl.ds(k * LANES, LANES)] = zeros
        return carry

    lax.fori_loop(0, CH, zero_body, 0)
    for k in range(stripe // CH):
        pltpu.sync_copy(zbuf, acc.at[pl.ds(s * stripe + k * CH, CH)])
    plsc.subcore_barrier()

    def edge_body(j, carry):
        pltpu.async_copy(g_sh.at[idx_s.at[j]], rows, gsem).wait()
        pltpu.sync_copy(rows, acc.at[idx_d.at[j]], add=True)
        return carry

    lax.fori_loop(0, nch, edge_body, 0)
    plsc.subcore_barrier()
    pltpu.sync_copy(acc.at[pl.ds(s * stripe, stripe)],
                    out_hbm.at[c, pl.ds(s * stripe, stripe)])


def _sc_scatter(g, src3, dst3):
    np_, h = g.shape
    nw, nch, ch = src3.shape
    assert nw == NW and ch == CH
    mesh = plsc.VectorSubcoreMesh(core_axis_name="c", subcore_axis_name="s",
                                  num_cores=NC)
    return pl.kernel(
        functools.partial(_scatter_body, np_, nch, h),
        out_type=jax.ShapeDtypeStruct((NC, np_, h), jnp.float32),
        mesh=mesh,
        scratch_types=[
            pltpu.VMEM((nch, CH), jnp.int32),
            pltpu.VMEM((nch, CH), jnp.int32),
            pltpu.VMEM((CH, h), jnp.float32),
            pltpu.VMEM((CH, h), jnp.float32),
            pltpu.VMEM_SHARED((np_, h), jnp.float32),
            pltpu.VMEM_SHARED((np_, h), jnp.float32),
            pltpu.SemaphoreType.DMA,
        ],
        compiler_params=pltpu.CompilerParams(needs_layout_passes=False,
                                             use_tc_tiling_on_sc=False),
    )(g, src3, dst3)


def _k2_body(x_ref, cnt_ref, wc_ref, c0_ref, g1_ref, dinv_ref):
    cnt = cnt_ref[0, :, 0] + cnt_ref[1, :, 0] + 1.0
    dinv = lax.rsqrt(cnt)[:, None]
    m = jnp.dot(x_ref[...], wc_ref[...],
                preferred_element_type=jnp.float32,
                precision=lax.Precision.HIGHEST) + c0_ref[...]
    g1_ref[...] = m * dinv
    dinv_ref[...] = dinv


def _layer_norm(t, g, b):
    mu = jnp.mean(t, axis=-1, keepdims=True)
    var = jnp.mean((t - mu) ** 2, axis=-1, keepdims=True)
    return (t - mu) / jnp.sqrt(var + 1e-5) * g + b


def _k4_body(p_ref, g1_ref, dinv_ref, b1_ref, lg_ref, lb_ref, w2_ref,
             prev_ref, g2_ref):
    dinv = dinv_ref[...]
    t = (p_ref[0] + p_ref[1] + g1_ref[...]) * dinv + b1_ref[...]
    ln = _layer_norm(t, lg_ref[...], lb_ref[...])
    prev_ref[...] = ln
    hr = jnp.maximum(ln, 0.0)
    g2_ref[...] = jnp.dot(hr, w2_ref[...],
                          preferred_element_type=jnp.float32,
                precision=lax.Precision.HIGHEST) * dinv


def _k6_body(q_ref, g2_ref, prev_ref, dinv_ref, b2_ref, lg_ref, lb_ref,
             wr1_ref, br1_ref, wr2_ref, br2_ref, y_ref):
    t = (q_ref[0] + q_ref[1] + g2_ref[...]) * dinv_ref[...] + b2_ref[...]
    ln = _layer_norm(t, lg_ref[...], lb_ref[...])
    hcat = ln + prev_ref[...]
    r = jnp.maximum(jnp.dot(hcat, wr1_ref[...],
                            preferred_element_type=jnp.float32,
                precision=lax.Precision.HIGHEST)
                    + br1_ref[...], 0.0)
    y = jnp.dot(r, wr2_ref[...],
                preferred_element_type=jnp.float32,
                precision=lax.Precision.HIGHEST) + br2_ref[...]
    y_ref[...] = jax.nn.sigmoid(y)


def _row_spec(bn, width):
    return pl.BlockSpec((bn, width), lambda i: (i, 0))


def _full_spec(shape):
    return pl.BlockSpec(shape, lambda i: tuple(0 for _ in shape))


def kernel(x, edge_index, bn_gamma, bn_beta, W_enc, b_enc, W1, b1, ln1_g,
           ln1_b, W2, b2, ln2_g, ln2_b, Wr1, br1, Wr2, br2):
    n, d = x.shape
    h = W1.shape[1]
    e = edge_index.shape[1]

    npad = ((n + 1 + 2047) // 2048) * 2048
    ew = ((e + NW * CH - 1) // (NW * CH)) * CH
    ep = ew * NW

    gscale = bn_gamma / jnp.sqrt(1.0 + 1e-5)
    wep = gscale[:, None] * W_enc
    bep = bn_beta @ W_enc + b_enc
    w1a, w1b = W1[:d], W1[d:]
    wc = w1a + wep @ w1b
    c0 = (bep @ w1b)[None, :]

    xpad = jnp.pad(x, ((0, npad - n), (0, 0)))
    src = jnp.concatenate([edge_index[0], jnp.full((ep - e,), n, jnp.int32)])
    dst = jnp.concatenate([edge_index[1], jnp.full((ep - e,), n, jnp.int32)])
    src3 = src.reshape(NW, ew // CH, CH)
    dst3 = dst.reshape(NW, ew // CH, CH)

    ones8 = jnp.zeros((CH, 8), jnp.float32).at[:, 0].set(1.0)
    zeros8 = jnp.zeros((npad // NS, 8), jnp.float32)
    cnt = _sc_count(dst3, ones8, zeros8, npad)

    bn = 512
    grid = (npad // bn,)
    g1, dinv = pl.pallas_call(
        _k2_body,
        grid=grid,
        in_specs=[
            _row_spec(bn, d),
            pl.BlockSpec((NC, bn, 8), lambda i: (0, i, 0)),
            _full_spec((d, h)),
            _full_spec((1, h)),
        ],
        out_specs=[_row_spec(bn, h), _row_spec(bn, 1)],
        out_shape=[
            jax.ShapeDtypeStruct((npad, h), jnp.float32),
            jax.ShapeDtypeStruct((npad, 1), jnp.float32),
        ],
    )(xpad, cnt, wc, c0)

    p = _sc_scatter(g1, src3, dst3)

    prev, g2 = pl.pallas_call(
        _k4_body,
        grid=grid,
        in_specs=[
            pl.BlockSpec((NC, bn, h), lambda i: (0, i, 0)),
            _row_spec(bn, h),
            _row_spec(bn, 1),
            _full_spec((1, h)),
            _full_spec((1, h)),
            _full_spec((1, h)),
            _full_spec((h, h)),
        ],
        out_specs=[_row_spec(bn, h), _row_spec(bn, h)],
        out_shape=[
            jax.ShapeDtypeStruct((npad, h), jnp.float32),
            jax.ShapeDtypeStruct((npad, h), jnp.float32),
        ],
    )(p, g1, dinv, b1[None, :], ln1_g[None, :], ln1_b[None, :], W2)

    q = _sc_scatter(g2, src3, dst3)

    hh = Wr1.shape[1]
    y = pl.pallas_call(
        _k6_body,
        grid=grid,
        in_specs=[
            pl.BlockSpec((NC, bn, h), lambda i: (0, i, 0)),
            _row_spec(bn, h),
            _row_spec(bn, h),
            _row_spec(bn, 1),
            _full_spec((1, h)),
            _full_spec((1, h)),
            _full_spec((1, h)),
            _full_spec((h, hh)),
            _full_spec((1, hh)),
            _full_spec((hh, 1)),
            _full_spec((1, 1)),
        ],
        out_specs=_row_spec(bn, 1),
        out_shape=jax.ShapeDtypeStruct((npad, 1), jnp.float32),
    )(q, g2, prev, dinv, b2[None, :], ln2_g[None, :], ln2_b[None, :],
      Wr1, br1[None, :], Wr2, br2[None, :])

    return y[:n]


# speedup vs baseline: 27.6980x; 1.1686x over previous
import functools

import jax
import jax.numpy as jnp
from jax import lax
from jax.experimental import pallas as pl
from jax.experimental.pallas import tpu as pltpu
from jax.experimental.pallas import tpu_sc as plsc

NC = 2
NS = 16
NW = NC * NS
LANES = 16
CH = 128


def _count_body(np_, nch, dst_hbm, ones_hbm, zeros_hbm, cnt_out,
                idx_d, ones_v, cnt_sh):
    c = lax.axis_index("c")
    s = lax.axis_index("s")
    w = c * NS + s
    stripe = np_ // NS
    pltpu.sync_copy(dst_hbm.at[w], idx_d)
    pltpu.sync_copy(ones_hbm, ones_v)
    pltpu.sync_copy(zeros_hbm, cnt_sh.at[pl.ds(s * stripe, stripe)])
    plsc.subcore_barrier()

    def count_body(j, carry):
        pltpu.sync_copy(ones_v, cnt_sh.at[idx_d.at[j]], add=True)
        return carry

    lax.fori_loop(0, nch, count_body, 0)
    plsc.subcore_barrier()
    pltpu.sync_copy(cnt_sh.at[pl.ds(s * stripe, stripe)],
                    cnt_out.at[c, pl.ds(s * stripe, stripe)])


def _sc_count(dst3, ones8, zeros8, np_):
    nw, nch, ch = dst3.shape
    assert nw == NW and ch == CH
    mesh = plsc.VectorSubcoreMesh(core_axis_name="c", subcore_axis_name="s",
                                  num_cores=NC)
    return pl.kernel(
        functools.partial(_count_body, np_, nch),
        out_type=jax.ShapeDtypeStruct((NC, np_, 8), jnp.float32),
        mesh=mesh,
        scratch_types=[
            pltpu.VMEM((nch, CH), jnp.int32),
            pltpu.VMEM((CH, 8), jnp.float32),
            pltpu.VMEM_SHARED((np_, 8), jnp.float32),
        ],
        compiler_params=pltpu.CompilerParams(needs_layout_passes=False,
                                             use_tc_tiling_on_sc=False),
    )(dst3, ones8, zeros8)


def _scatter_body(np_, nch, h, g_hbm, src_hbm, dst_hbm, out_hbm,
                  idx_s, idx_d, rows0, rows1, g_sh, acc, gsem0, gsem1):
    c = lax.axis_index("c")
    s = lax.axis_index("s")
    w = c * NS + s
    stripe = np_ // NS

    pltpu.sync_copy(src_hbm.at[w], idx_s)
    pltpu.sync_copy(dst_hbm.at[w], idx_d)

    pltpu.sync_copy(g_hbm.at[pl.ds(s * stripe, stripe)],
                    g_sh.at[pl.ds(s * stripe, stripe)])
    zeros = jnp.zeros((LANES,), jnp.float32)

    def zero_body(i, carry):
        for k in range(h // LANES):
            rows0[i, pl.ds(k * LANES, LANES)] = zeros
        return carry

    lax.fori_loop(0, CH, zero_body, 0)
    for k in range(stripe // CH):
        pltpu.sync_copy(rows0, acc.at[pl.ds(s * stripe + k * CH, CH)])
    plsc.subcore_barrier()

    pltpu.make_async_copy(g_sh.at[idx_s.at[0]], rows0, gsem0).start()
    npairs = nch // 2

    def edge_pair(i, carry):
        j0 = 2 * i
        pltpu.make_async_copy(g_sh.at[idx_s.at[j0 + 1]], rows1,
                              gsem1).start()
        pltpu.make_async_copy(g_sh.at[idx_s.at[j0]], rows0, gsem0).wait()
        pltpu.sync_copy(rows0, acc.at[idx_d.at[j0]], add=True)

        @pl.when(i + 1 < npairs)
        def _():
            pltpu.make_async_copy(g_sh.at[idx_s.at[j0 + 2]], rows0,
                                  gsem0).start()

        pltpu.make_async_copy(g_sh.at[idx_s.at[j0 + 1]], rows1,
                              gsem1).wait()
        pltpu.sync_copy(rows1, acc.at[idx_d.at[j0 + 1]], add=True)
        return carry

    lax.fori_loop(0, npairs, edge_pair, 0)
    plsc.subcore_barrier()
    pltpu.sync_copy(acc.at[pl.ds(s * stripe, stripe)],
                    out_hbm.at[c, pl.ds(s * stripe, stripe)])


def _sc_scatter(g, src3, dst3):
    np_, h = g.shape
    nw, nch, ch = src3.shape
    assert nw == NW and ch == CH
    mesh = plsc.VectorSubcoreMesh(core_axis_name="c", subcore_axis_name="s",
                                  num_cores=NC)
    return pl.kernel(
        functools.partial(_scatter_body, np_, nch, h),
        out_type=jax.ShapeDtypeStruct((NC, np_, h), jnp.float32),
        mesh=mesh,
        scratch_types=[
            pltpu.VMEM((nch, CH), jnp.int32),
            pltpu.VMEM((nch, CH), jnp.int32),
            pltpu.VMEM((CH, h), jnp.float32),
            pltpu.VMEM((CH, h), jnp.float32),
            pltpu.VMEM_SHARED((np_, h), jnp.float32),
            pltpu.VMEM_SHARED((np_, h), jnp.float32),
            pltpu.SemaphoreType.DMA,
            pltpu.SemaphoreType.DMA,
        ],
        compiler_params=pltpu.CompilerParams(needs_layout_passes=False,
                                             use_tc_tiling_on_sc=False),
    )(g, src3, dst3)


def _k2a_body(x_ref, wc_ref, c0_ref, m_ref):
    m_ref[...] = jnp.dot(x_ref[...], wc_ref[...],
                         preferred_element_type=jnp.float32,
                         precision=lax.Precision.HIGHEST) + c0_ref[...]


def _k2b_body(m_ref, cnt_ref, g1_ref, dinv_ref):
    cnt = cnt_ref[0, :, 0] + cnt_ref[1, :, 0] + 1.0
    dinv = lax.rsqrt(cnt)[:, None]
    g1_ref[...] = m_ref[...] * dinv
    dinv_ref[...] = dinv


def _layer_norm(t, g, b):
    mu = jnp.mean(t, axis=-1, keepdims=True)
    var = jnp.mean((t - mu) ** 2, axis=-1, keepdims=True)
    return (t - mu) / jnp.sqrt(var + 1e-5) * g + b


def _k4_body(p_ref, g1_ref, dinv_ref, b1_ref, lg_ref, lb_ref, w2_ref,
             prev_ref, g2_ref):
    dinv = dinv_ref[...]
    t = (p_ref[0] + p_ref[1] + g1_ref[...]) * dinv + b1_ref[...]
    ln = _layer_norm(t, lg_ref[...], lb_ref[...])
    prev_ref[...] = ln
    hr = jnp.maximum(ln, 0.0)
    g2_ref[...] = jnp.dot(hr, w2_ref[...],
                          preferred_element_type=jnp.float32,
                precision=lax.Precision.HIGHEST) * dinv


def _k6_body(q_ref, g2_ref, prev_ref, dinv_ref, b2_ref, lg_ref, lb_ref,
             wr1_ref, br1_ref, wr2_ref, br2_ref, y_ref):
    t = (q_ref[0] + q_ref[1] + g2_ref[...]) * dinv_ref[...] + b2_ref[...]
    ln = _layer_norm(t, lg_ref[...], lb_ref[...])
    hcat = ln + prev_ref[...]
    r = jnp.maximum(jnp.dot(hcat, wr1_ref[...],
                            preferred_element_type=jnp.float32,
                precision=lax.Precision.HIGHEST)
                    + br1_ref[...], 0.0)
    y = jnp.dot(r, wr2_ref[...],
                preferred_element_type=jnp.float32,
                precision=lax.Precision.HIGHEST) + br2_ref[...]
    y_ref[...] = jax.nn.sigmoid(y)


def _row_spec(bn, width):
    return pl.BlockSpec((bn, width), lambda i: (i, 0))


def _full_spec(shape):
    return pl.BlockSpec(shape, lambda i: tuple(0 for _ in shape))


def kernel(x, edge_index, bn_gamma, bn_beta, W_enc, b_enc, W1, b1, ln1_g,
           ln1_b, W2, b2, ln2_g, ln2_b, Wr1, br1, Wr2, br2):
    n, d = x.shape
    h = W1.shape[1]
    e = edge_index.shape[1]

    npad = ((n + 1 + 2047) // 2048) * 2048
    ew = ((e + NW * 2 * CH - 1) // (NW * 2 * CH)) * 2 * CH
    ep = ew * NW

    gscale = bn_gamma / jnp.sqrt(1.0 + 1e-5)
    wep = gscale[:, None] * W_enc
    bep = bn_beta @ W_enc + b_enc
    w1a, w1b = W1[:d], W1[d:]
    wc = w1a + wep @ w1b
    c0 = (bep @ w1b)[None, :]

    xpad = jnp.pad(x, ((0, npad - n), (0, 0)))
    src = jnp.concatenate([edge_index[0], jnp.full((ep - e,), n, jnp.int32)])
    dst = jnp.concatenate([edge_index[1], jnp.full((ep - e,), n, jnp.int32)])
    src3 = src.reshape(NW, ew // CH, CH)
    dst3 = dst.reshape(NW, ew // CH, CH)

    ones8 = jnp.zeros((CH, 8), jnp.float32).at[:, 0].set(1.0)
    zeros8 = jnp.zeros((npad // NS, 8), jnp.float32)
    cnt = _sc_count(dst3, ones8, zeros8, npad)

    bn = 512
    grid = (npad // bn,)
    m = pl.pallas_call(
        _k2a_body,
        grid=grid,
        in_specs=[_row_spec(bn, d), _full_spec((d, h)), _full_spec((1, h))],
        out_specs=_row_spec(bn, h),
        out_shape=jax.ShapeDtypeStruct((npad, h), jnp.float32),
    )(xpad, wc, c0)

    g1, dinv = pl.pallas_call(
        _k2b_body,
        grid=grid,
        in_specs=[
            _row_spec(bn, h),
            pl.BlockSpec((NC, bn, 8), lambda i: (0, i, 0)),
        ],
        out_specs=[_row_spec(bn, h), _row_spec(bn, 1)],
        out_shape=[
            jax.ShapeDtypeStruct((npad, h), jnp.float32),
            jax.ShapeDtypeStruct((npad, 1), jnp.float32),
        ],
    )(m, cnt)

    p = _sc_scatter(g1, src3, dst3)

    prev, g2 = pl.pallas_call(
        _k4_body,
        grid=grid,
        in_specs=[
            pl.BlockSpec((NC, bn, h), lambda i: (0, i, 0)),
            _row_spec(bn, h),
            _row_spec(bn, 1),
            _full_spec((1, h)),
            _full_spec((1, h)),
            _full_spec((1, h)),
            _full_spec((h, h)),
        ],
        out_specs=[_row_spec(bn, h), _row_spec(bn, h)],
        out_shape=[
            jax.ShapeDtypeStruct((npad, h), jnp.float32),
            jax.ShapeDtypeStruct((npad, h), jnp.float32),
        ],
    )(p, g1, dinv, b1[None, :], ln1_g[None, :], ln1_b[None, :], W2)

    q = _sc_scatter(g2, src3, dst3)

    hh = Wr1.shape[1]
    y = pl.pallas_call(
        _k6_body,
        grid=grid,
        in_specs=[
            pl.BlockSpec((NC, bn, h), lambda i: (0, i, 0)),
            _row_spec(bn, h),
            _row_spec(bn, h),
            _row_spec(bn, 1),
            _full_spec((1, h)),
            _full_spec((1, h)),
            _full_spec((1, h)),
            _full_spec((h, hh)),
            _full_spec((1, hh)),
            _full_spec((hh, 1)),
            _full_spec((1, 1)),
        ],
        out_specs=_row_spec(bn, 1),
        out_shape=jax.ShapeDtypeStruct((npad, 1), jnp.float32),
    )(q, g2, prev, dinv, b2[None, :], ln2_g[None, :], ln2_b[None, :],
      Wr1, br1[None, :], Wr2, br2[None, :])

    return y[:n]


# speedup vs baseline: 30.8461x; 1.1137x over previous
import functools

import jax
import jax.numpy as jnp
from jax import lax
from jax.experimental import pallas as pl
from jax.experimental.pallas import tpu as pltpu
from jax.experimental.pallas import tpu_sc as plsc

NC = 2
NS = 16
NW = NC * NS
LANES = 16
CH = 128


def _count_body(np_, nch, dst_hbm, ones_hbm, zeros_hbm, cnt_out,
                idx_d, ones_v, cnt_sh):
    c = lax.axis_index("c")
    s = lax.axis_index("s")
    w = c * NS + s
    stripe = np_ // NS
    pltpu.sync_copy(dst_hbm.at[w], idx_d)
    pltpu.sync_copy(ones_hbm, ones_v)
    pltpu.sync_copy(zeros_hbm, cnt_sh.at[pl.ds(s * stripe, stripe)])
    plsc.subcore_barrier()

    def count_body(j, carry):
        pltpu.sync_copy(ones_v, cnt_sh.at[idx_d.at[j]], add=True)
        return carry

    lax.fori_loop(0, nch, count_body, 0)
    plsc.subcore_barrier()
    pltpu.sync_copy(cnt_sh.at[pl.ds(s * stripe, stripe)],
                    cnt_out.at[c, pl.ds(s * stripe, stripe)])


def _sc_count(dst3, ones8, zeros8, np_):
    nw, nch, ch = dst3.shape
    assert nw == NW and ch == CH
    mesh = plsc.VectorSubcoreMesh(core_axis_name="c", subcore_axis_name="s",
                                  num_cores=NC)
    return pl.kernel(
        functools.partial(_count_body, np_, nch),
        out_type=jax.ShapeDtypeStruct((NC, np_, 8), jnp.float32),
        mesh=mesh,
        scratch_types=[
            pltpu.VMEM((nch, CH), jnp.int32),
            pltpu.VMEM((CH, 8), jnp.float32),
            pltpu.VMEM_SHARED((np_, 8), jnp.float32),
        ],
        compiler_params=pltpu.CompilerParams(needs_layout_passes=False,
                                             use_tc_tiling_on_sc=False),
    )(dst3, ones8, zeros8)


def _scatter_body(np_, nch, h, g_hbm, src_hbm, dst_hbm, out_hbm,
                  idx_s, idx_d, rows0, rows1, g_sh, acc, gsem0, gsem1):
    c = lax.axis_index("c")
    s = lax.axis_index("s")
    w = c * NS + s
    stripe = np_ // NS

    pltpu.sync_copy(src_hbm.at[w], idx_s)
    pltpu.sync_copy(dst_hbm.at[w], idx_d)

    pltpu.sync_copy(g_hbm.at[pl.ds(s * stripe, stripe)],
                    g_sh.at[pl.ds(s * stripe, stripe)])
    zeros = jnp.zeros((LANES,), jnp.float32)

    def zero_body(i, carry):
        for k in range(h // LANES):
            rows0[i, pl.ds(k * LANES, LANES)] = zeros
        return carry

    lax.fori_loop(0, CH, zero_body, 0)
    for k in range(stripe // CH):
        pltpu.sync_copy(rows0, acc.at[pl.ds(s * stripe + k * CH, CH)])
    plsc.subcore_barrier()

    pltpu.make_async_copy(g_sh.at[idx_s.at[0]], rows0, gsem0).start()
    npairs = nch // 2

    def edge_pair(i, carry):
        j0 = 2 * i
        pltpu.make_async_copy(g_sh.at[idx_s.at[j0 + 1]], rows1,
                              gsem1).start()
        pltpu.make_async_copy(g_sh.at[idx_s.at[j0]], rows0, gsem0).wait()
        pltpu.sync_copy(rows0, acc.at[idx_d.at[j0]], add=True)

        @pl.when(i + 1 < npairs)
        def _():
            pltpu.make_async_copy(g_sh.at[idx_s.at[j0 + 2]], rows0,
                                  gsem0).start()

        pltpu.make_async_copy(g_sh.at[idx_s.at[j0 + 1]], rows1,
                              gsem1).wait()
        pltpu.sync_copy(rows1, acc.at[idx_d.at[j0 + 1]], add=True)
        return carry

    lax.fori_loop(0, npairs, edge_pair, 0)
    plsc.subcore_barrier()
    pltpu.sync_copy(acc.at[pl.ds(s * stripe, stripe)],
                    out_hbm.at[c, pl.ds(s * stripe, stripe)])


def _sc_scatter(g, src3, dst3):
    np_, h = g.shape
    nw, nch, ch = src3.shape
    assert nw == NW and ch == CH
    mesh = plsc.VectorSubcoreMesh(core_axis_name="c", subcore_axis_name="s",
                                  num_cores=NC)
    return pl.kernel(
        functools.partial(_scatter_body, np_, nch, h),
        out_type=jax.ShapeDtypeStruct((NC, np_, h), jnp.float32),
        mesh=mesh,
        scratch_types=[
            pltpu.VMEM((nch, CH), jnp.int32),
            pltpu.VMEM((nch, CH), jnp.int32),
            pltpu.VMEM((CH, h), jnp.float32),
            pltpu.VMEM((CH, h), jnp.float32),
            pltpu.VMEM_SHARED((np_, h), jnp.float32),
            pltpu.VMEM_SHARED((np_, h), jnp.float32),
            pltpu.SemaphoreType.DMA,
            pltpu.SemaphoreType.DMA,
        ],
        compiler_params=pltpu.CompilerParams(needs_layout_passes=False,
                                             use_tc_tiling_on_sc=False),
    )(g, src3, dst3)


def _k2a_body(x_ref, wc_ref, c0_ref, m_ref):
    m_ref[...] = jnp.dot(x_ref[...], wc_ref[...],
                         preferred_element_type=jnp.float32) + c0_ref[...]


def _k2b_body(m_ref, cnt_ref, g1_ref, dinv_ref):
    cnt = cnt_ref[0, :, 0] + cnt_ref[1, :, 0] + 1.0
    dinv = lax.rsqrt(cnt)[:, None]
    g1_ref[...] = m_ref[...] * dinv
    dinv_ref[...] = dinv


def _layer_norm(t, g, b):
    mu = jnp.mean(t, axis=-1, keepdims=True)
    var = jnp.mean((t - mu) ** 2, axis=-1, keepdims=True)
    return (t - mu) / jnp.sqrt(var + 1e-5) * g + b


def _k4_body(p_ref, g1_ref, dinv_ref, b1_ref, lg_ref, lb_ref, w2_ref,
             prev_ref, g2_ref):
    dinv = dinv_ref[...]
    t = (p_ref[0] + p_ref[1] + g1_ref[...]) * dinv + b1_ref[...]
    ln = _layer_norm(t, lg_ref[...], lb_ref[...])
    prev_ref[...] = ln
    hr = jnp.maximum(ln, 0.0)
    g2_ref[...] = jnp.dot(hr, w2_ref[...],
                          preferred_element_type=jnp.float32) * dinv


def _k6_body(q_ref, g2_ref, prev_ref, dinv_ref, b2_ref, lg_ref, lb_ref,
             wr1_ref, br1_ref, wr2_ref, br2_ref, y_ref):
    t = (q_ref[0] + q_ref[1] + g2_ref[...]) * dinv_ref[...] + b2_ref[...]
    ln = _layer_norm(t, lg_ref[...], lb_ref[...])
    hcat = ln + prev_ref[...]
    r = jnp.maximum(jnp.dot(hcat, wr1_ref[...],
                            preferred_element_type=jnp.float32)
                    + br1_ref[...], 0.0)
    y = jnp.sum(r * wr2_ref[...].T, axis=-1, keepdims=True) + br2_ref[...]
    y_ref[...] = jax.nn.sigmoid(y)


def _row_spec(bn, width):
    return pl.BlockSpec((bn, width), lambda i: (i, 0))


def _full_spec(shape):
    return pl.BlockSpec(shape, lambda i: tuple(0 for _ in shape))


def kernel(x, edge_index, bn_gamma, bn_beta, W_enc, b_enc, W1, b1, ln1_g,
           ln1_b, W2, b2, ln2_g, ln2_b, Wr1, br1, Wr2, br2):
    n, d = x.shape
    h = W1.shape[1]
    e = edge_index.shape[1]

    npad = ((n + 1 + 2047) // 2048) * 2048
    ew = ((e + NW * 2 * CH - 1) // (NW * 2 * CH)) * 2 * CH
    ep = ew * NW

    gscale = bn_gamma / jnp.sqrt(1.0 + 1e-5)
    wep = gscale[:, None] * W_enc
    bep = bn_beta @ W_enc + b_enc
    w1a, w1b = W1[:d], W1[d:]
    wc = w1a + wep @ w1b
    c0 = (bep @ w1b)[None, :]

    src = jnp.concatenate([edge_index[0], jnp.full((ep - e,), n, jnp.int32)])
    dst = jnp.concatenate([edge_index[1], jnp.full((ep - e,), n, jnp.int32)])
    src3 = src.reshape(NW, ew // CH, CH)
    dst3 = dst.reshape(NW, ew // CH, CH)

    ones8 = jnp.zeros((CH, 8), jnp.float32).at[:, 0].set(1.0)
    zeros8 = jnp.zeros((npad // NS, 8), jnp.float32)
    cnt = _sc_count(dst3, ones8, zeros8, npad)

    bn = 1024
    grid = (npad // bn,)
    m = pl.pallas_call(
        _k2a_body,
        grid=grid,
        in_specs=[_row_spec(bn, d), _full_spec((d, h)), _full_spec((1, h))],
        out_specs=_row_spec(bn, h),
        out_shape=jax.ShapeDtypeStruct((npad, h), jnp.float32),
    )(x, wc, c0)

    g1, dinv = pl.pallas_call(
        _k2b_body,
        grid=grid,
        in_specs=[
            _row_spec(bn, h),
            pl.BlockSpec((NC, bn, 8), lambda i: (0, i, 0)),
        ],
        out_specs=[_row_spec(bn, h), _row_spec(bn, 1)],
        out_shape=[
            jax.ShapeDtypeStruct((npad, h), jnp.float32),
            jax.ShapeDtypeStruct((npad, 1), jnp.float32),
        ],
    )(m, cnt)

    p = _sc_scatter(g1, src3, dst3)

    prev, g2 = pl.pallas_call(
        _k4_body,
        grid=grid,
        in_specs=[
            pl.BlockSpec((NC, bn, h), lambda i: (0, i, 0)),
            _row_spec(bn, h),
            _row_spec(bn, 1),
            _full_spec((1, h)),
            _full_spec((1, h)),
            _full_spec((1, h)),
            _full_spec((h, h)),
        ],
        out_specs=[_row_spec(bn, h), _row_spec(bn, h)],
        out_shape=[
            jax.ShapeDtypeStruct((npad, h), jnp.float32),
            jax.ShapeDtypeStruct((npad, h), jnp.float32),
        ],
    )(p, g1, dinv, b1[None, :], ln1_g[None, :], ln1_b[None, :], W2)

    q = _sc_scatter(g2, src3, dst3)

    hh = Wr1.shape[1]
    y = pl.pallas_call(
        _k6_body,
        grid=grid,
        in_specs=[
            pl.BlockSpec((NC, bn, h), lambda i: (0, i, 0)),
            _row_spec(bn, h),
            _row_spec(bn, h),
            _row_spec(bn, 1),
            _full_spec((1, h)),
            _full_spec((1, h)),
            _full_spec((1, h)),
            _full_spec((h, hh)),
            _full_spec((1, hh)),
            _full_spec((hh, 1)),
            _full_spec((1, 1)),
        ],
        out_specs=_row_spec(bn, 1),
        out_shape=jax.ShapeDtypeStruct((npad, 1), jnp.float32),
    )(q, g2, prev, dinv, b2[None, :], ln2_g[None, :], ln2_b[None, :],
      Wr1, br1[None, :], Wr2, br2[None, :])

    return y[:n]


# speedup vs baseline: 32.6976x; 1.0600x over previous
import functools

import jax
import jax.numpy as jnp
from jax import lax
from jax.experimental import pallas as pl
from jax.experimental.pallas import tpu as pltpu
from jax.experimental.pallas import tpu_sc as plsc

NC = 2
NS = 16
NW = NC * NS
LANES = 16
CH = 128


def _load_chunks(e3, row, base, rem, w, idx_v):
    start = w * base + jnp.minimum(w, rem)
    pltpu.sync_copy(e3.at[row, pl.ds(start, base)], idx_v.at[pl.ds(0, base)])
    if rem:
        @pl.when(w < rem)
        def _():
            pltpu.sync_copy(e3.at[row, start + base], idx_v.at[base])


def _count_body(np_, base, rem, e3_hbm, ones_hbm, zeros_hbm, cnt_out,
                idx_d, ones_v, cnt_sh):
    c = lax.axis_index("c")
    s = lax.axis_index("s")
    w = c * NS + s
    stripe = np_ // NS
    _load_chunks(e3_hbm, 1, base, rem, w, idx_d)
    pltpu.sync_copy(ones_hbm, ones_v)
    pltpu.sync_copy(zeros_hbm, cnt_sh.at[pl.ds(s * stripe, stripe)])
    plsc.subcore_barrier()

    def count_body(j, carry):
        pltpu.sync_copy(ones_v, cnt_sh.at[idx_d.at[j]], add=True)
        return carry

    lax.fori_loop(0, base, count_body, 0)
    if rem:
        @pl.when(w < rem)
        def _():
            count_body(base, 0)
    plsc.subcore_barrier()
    pltpu.sync_copy(cnt_sh.at[pl.ds(s * stripe, stripe)],
                    cnt_out.at[c, pl.ds(s * stripe, stripe)])


def _sc_count(e3, ones8, zeros8, np_):
    two, tch, ch = e3.shape
    assert two == 2 and ch == CH
    base, rem = tch // NW, tch % NW
    nmax = base + (1 if rem else 0)
    mesh = plsc.VectorSubcoreMesh(core_axis_name="c", subcore_axis_name="s",
                                  num_cores=NC)
    return pl.kernel(
        functools.partial(_count_body, np_, base, rem),
        out_type=jax.ShapeDtypeStruct((NC, np_, 8), jnp.float32),
        mesh=mesh,
        scratch_types=[
            pltpu.VMEM((nmax, CH), jnp.int32),
            pltpu.VMEM((CH, 8), jnp.float32),
            pltpu.VMEM_SHARED((np_, 8), jnp.float32),
        ],
        compiler_params=pltpu.CompilerParams(needs_layout_passes=False,
                                             use_tc_tiling_on_sc=False),
    )(e3, ones8, zeros8)


def _scatter_body(np_, base, rem, h, g_hbm, e3_hbm, out_hbm,
                  idx_s, idx_d, rows0, rows1, g_sh, acc, gsem0, gsem1):
    c = lax.axis_index("c")
    s = lax.axis_index("s")
    w = c * NS + s
    stripe = np_ // NS

    _load_chunks(e3_hbm, 0, base, rem, w, idx_s)
    _load_chunks(e3_hbm, 1, base, rem, w, idx_d)

    pltpu.sync_copy(g_hbm.at[pl.ds(s * stripe, stripe)],
                    g_sh.at[pl.ds(s * stripe, stripe)])
    zeros = jnp.zeros((LANES,), jnp.float32)

    def zero_body(i, carry):
        for k in range(h // LANES):
            rows0[i, pl.ds(k * LANES, LANES)] = zeros
        return carry

    lax.fori_loop(0, CH, zero_body, 0)
    for k in range(stripe // CH):
        pltpu.sync_copy(rows0, acc.at[pl.ds(s * stripe + k * CH, CH)])
    plsc.subcore_barrier()

    pltpu.make_async_copy(g_sh.at[idx_s.at[0]], rows0, gsem0).start()
    npairs = base // 2

    def edge_pair(i, carry):
        j0 = 2 * i
        pltpu.make_async_copy(g_sh.at[idx_s.at[j0 + 1]], rows1,
                              gsem1).start()
        pltpu.make_async_copy(g_sh.at[idx_s.at[j0]], rows0, gsem0).wait()
        pltpu.sync_copy(rows0, acc.at[idx_d.at[j0]], add=True)

        @pl.when(i + 1 < npairs)
        def _():
            pltpu.make_async_copy(g_sh.at[idx_s.at[j0 + 2]], rows0,
                                  gsem0).start()

        pltpu.make_async_copy(g_sh.at[idx_s.at[j0 + 1]], rows1,
                              gsem1).wait()
        pltpu.sync_copy(rows1, acc.at[idx_d.at[j0 + 1]], add=True)
        return carry

    lax.fori_loop(0, npairs, edge_pair, 0)
    if rem:
        @pl.when(w < rem)
        def _():
            pltpu.async_copy(g_sh.at[idx_s.at[base]], rows0, gsem0).wait()
            pltpu.sync_copy(rows0, acc.at[idx_d.at[base]], add=True)
    plsc.subcore_barrier()
    pltpu.sync_copy(acc.at[pl.ds(s * stripe, stripe)],
                    out_hbm.at[c, pl.ds(s * stripe, stripe)])


def _sc_scatter(g, e3):
    np_, h = g.shape
    two, tch, ch = e3.shape
    assert two == 2 and ch == CH
    base, rem = tch // NW, tch % NW
    assert base % 2 == 0
    nmax = base + (1 if rem else 0)
    mesh = plsc.VectorSubcoreMesh(core_axis_name="c", subcore_axis_name="s",
                                  num_cores=NC)
    return pl.kernel(
        functools.partial(_scatter_body, np_, base, rem, h),
        out_type=jax.ShapeDtypeStruct((NC, np_, h), jnp.float32),
        mesh=mesh,
        scratch_types=[
            pltpu.VMEM((nmax, CH), jnp.int32),
            pltpu.VMEM((nmax, CH), jnp.int32),
            pltpu.VMEM((CH, h), jnp.float32),
            pltpu.VMEM((CH, h), jnp.float32),
            pltpu.VMEM_SHARED((np_, h), jnp.float32),
            pltpu.VMEM_SHARED((np_, h), jnp.float32),
            pltpu.SemaphoreType.DMA,
            pltpu.SemaphoreType.DMA,
        ],
        compiler_params=pltpu.CompilerParams(needs_layout_passes=False,
                                             use_tc_tiling_on_sc=False),
    )(g, e3)


def _k2a_body(x_ref, wc_ref, c0_ref, m_ref):
    m_ref[...] = jnp.dot(x_ref[...], wc_ref[...],
                         preferred_element_type=jnp.float32) + c0_ref[...]


def _k2b_body(m_ref, cnt_ref, g1_ref, dinv_ref):
    cnt = cnt_ref[0, :, 0] + cnt_ref[1, :, 0] + 1.0
    dinv = lax.rsqrt(cnt)[:, None]
    g1_ref[...] = m_ref[...] * dinv
    dinv_ref[...] = dinv


def _layer_norm(t, g, b):
    mu = jnp.mean(t, axis=-1, keepdims=True)
    var = jnp.mean((t - mu) ** 2, axis=-1, keepdims=True)
    return (t - mu) / jnp.sqrt(var + 1e-5) * g + b


def _k4_body(p_ref, g1_ref, dinv_ref, b1_ref, lg_ref, lb_ref, w2_ref,
             prev_ref, g2_ref):
    dinv = dinv_ref[...]
    t = (p_ref[0] + p_ref[1] + g1_ref[...]) * dinv + b1_ref[...]
    ln = _layer_norm(t, lg_ref[...], lb_ref[...])
    prev_ref[...] = ln
    hr = jnp.maximum(ln, 0.0)
    g2_ref[...] = jnp.dot(hr, w2_ref[...],
                          preferred_element_type=jnp.float32) * dinv


def _k6_body(q_ref, g2_ref, prev_ref, dinv_ref, b2_ref, lg_ref, lb_ref,
             wr1_ref, br1_ref, wr2_ref, br2_ref, y_ref):
    t = (q_ref[0] + q_ref[1] + g2_ref[...]) * dinv_ref[...] + b2_ref[...]
    ln = _layer_norm(t, lg_ref[...], lb_ref[...])
    hcat = ln + prev_ref[...]
    r = jnp.maximum(jnp.dot(hcat, wr1_ref[...],
                            preferred_element_type=jnp.float32)
                    + br1_ref[...], 0.0)
    y = jnp.sum(r * wr2_ref[...].T, axis=-1, keepdims=True) + br2_ref[...]
    y_ref[...] = jax.nn.sigmoid(y)


def _row_spec(bn, width):
    return pl.BlockSpec((bn, width), lambda i: (i, 0))


def _full_spec(shape):
    return pl.BlockSpec(shape, lambda i: tuple(0 for _ in shape))


def kernel(x, edge_index, bn_gamma, bn_beta, W_enc, b_enc, W1, b1, ln1_g,
           ln1_b, W2, b2, ln2_g, ln2_b, Wr1, br1, Wr2, br2):
    n, d = x.shape
    h = W1.shape[1]
    e = edge_index.shape[1]

    npad = ((n + 1 + 2047) // 2048) * 2048
    assert e % CH == 0

    gscale = bn_gamma / jnp.sqrt(1.0 + 1e-5)
    wep = gscale[:, None] * W_enc
    bep = bn_beta @ W_enc + b_enc
    w1a, w1b = W1[:d], W1[d:]
    wc = w1a + wep @ w1b
    c0 = (bep @ w1b)[None, :]

    e3 = edge_index.reshape(2, e // CH, CH)

    ones8 = jnp.zeros((CH, 8), jnp.float32).at[:, 0].set(1.0)
    zeros8 = jnp.zeros((npad // NS, 8), jnp.float32)
    cnt = _sc_count(e3, ones8, zeros8, npad)

    bn = 1024
    grid = (npad // bn,)
    m = pl.pallas_call(
        _k2a_body,
        grid=grid,
        in_specs=[_row_spec(bn, d), _full_spec((d, h)), _full_spec((1, h))],
        out_specs=_row_spec(bn, h),
        out_shape=jax.ShapeDtypeStruct((npad, h), jnp.float32),
    )(x, wc, c0)

    g1, dinv = pl.pallas_call(
        _k2b_body,
        grid=grid,
        in_specs=[
            _row_spec(bn, h),
            pl.BlockSpec((NC, bn, 8), lambda i: (0, i, 0)),
        ],
        out_specs=[_row_spec(bn, h), _row_spec(bn, 1)],
        out_shape=[
            jax.ShapeDtypeStruct((npad, h), jnp.float32),
            jax.ShapeDtypeStruct((npad, 1), jnp.float32),
        ],
    )(m, cnt)

    p = _sc_scatter(g1, e3)

    prev, g2 = pl.pallas_call(
        _k4_body,
        grid=grid,
        in_specs=[
            pl.BlockSpec((NC, bn, h), lambda i: (0, i, 0)),
            _row_spec(bn, h),
            _row_spec(bn, 1),
            _full_spec((1, h)),
            _full_spec((1, h)),
            _full_spec((1, h)),
            _full_spec((h, h)),
        ],
        out_specs=[_row_spec(bn, h), _row_spec(bn, h)],
        out_shape=[
            jax.ShapeDtypeStruct((npad, h), jnp.float32),
            jax.ShapeDtypeStruct((npad, h), jnp.float32),
        ],
    )(p, g1, dinv, b1[None, :], ln1_g[None, :], ln1_b[None, :], W2)

    q = _sc_scatter(g2, e3)

    hh = Wr1.shape[1]
    y = pl.pallas_call(
        _k6_body,
        grid=grid,
        in_specs=[
            pl.BlockSpec((NC, bn, h), lambda i: (0, i, 0)),
            _row_spec(bn, h),
            _row_spec(bn, h),
            _row_spec(bn, 1),
            _full_spec((1, h)),
            _full_spec((1, h)),
            _full_spec((1, h)),
            _full_spec((h, hh)),
            _full_spec((1, hh)),
            _full_spec((hh, 1)),
            _full_spec((1, 1)),
        ],
        out_specs=_row_spec(bn, 1),
        out_shape=jax.ShapeDtypeStruct((npad, 1), jnp.float32),
    )(q, g2, prev, dinv, b2[None, :], ln2_g[None, :], ln2_b[None, :],
      Wr1, br1[None, :], Wr2, br2[None, :])

    return y[:n]
